# baseline (speedup 1.0000x reference)
import functools

import jax
import jax.numpy as jnp
from jax import lax
from jax.experimental import pallas as pl
from jax.experimental.pallas import tpu as pltpu
from jax.experimental.pallas import tpu_sc as plsc

N_REGS = 32
BIT_WIDTH = 64
KEY_DIM = 128
BATCH = 16384

_NC = 2
_NS = 16
_NW = _NC * _NS
_BPW = BATCH // _NW
_CHUNK = 128
_NCH = _BPW // _CHUNK


def _table_body(keys_ref, w1_ref, b1_ref, w2_ref, b2_ref, vals_ref, out_ref):
    r = lax.broadcasted_iota(jnp.int32, (N_REGS, 8), 0)
    c = lax.broadcasted_iota(jnp.int32, (N_REGS, 8), 1)
    bits = jnp.where(c < 5, (r >> c) & 1, 0).astype(jnp.float32)
    h = jnp.dot(bits, w1_ref[...], preferred_element_type=jnp.float32) + b1_ref[...]
    h = 0.5 * h * (1.0 + lax.erf(h * (2.0 ** -0.5)))
    q = jnp.dot(h, w2_ref[...], preferred_element_type=jnp.float32) + b2_ref[...]
    sim = lax.dot_general(q, keys_ref[...], (((1,), (1,)), ((), ())),
                          preferred_element_type=jnp.float32)
    m = jnp.max(sim, axis=1, keepdims=True)
    e = jnp.exp(sim - m)
    attn = e / jnp.sum(e, axis=1, keepdims=True)
    tab = jnp.dot(attn, vals_ref[...], preferred_element_type=jnp.float32)
    row = lax.broadcasted_iota(jnp.int32, (N_REGS, BIT_WIDTH), 0)
    tab = jnp.where(row == N_REGS - 1, 0.0, tab)
    out_ref[...] = jnp.concatenate(
        [tab, jnp.zeros((N_REGS, BIT_WIDTH), jnp.float32)], axis=1)


def _build_table(keys_scaled, w1p, b1, w2, b2, vals):
    return pl.pallas_call(
        _table_body,
        out_shape=jax.ShapeDtypeStruct((N_REGS, 2 * BIT_WIDTH), jnp.float32),
    )(keys_scaled, w1p, b1, w2, b2, vals)


@functools.cache
def _gather_kernel():
    mesh = plsc.VectorSubcoreMesh(core_axis_name="c", subcore_axis_name="s")

    @functools.partial(
        pl.kernel,
        mesh=mesh,
        out_type=jax.ShapeDtypeStruct((BATCH, BIT_WIDTH), jnp.float32),
        scratch_types=[
            pltpu.VMEM((_BPW,), jnp.int32),
            pltpu.VMEM((_CHUNK, 2 * BIT_WIDTH), jnp.float32),
            pltpu.VMEM((_CHUNK, 2 * BIT_WIDTH), jnp.float32),
            pltpu.VMEM((_BPW, BIT_WIDTH), jnp.float32),
            pltpu.SemaphoreType.DMA,
            pltpu.SemaphoreType.DMA,
        ],
    )
    def _gather(table_hbm, idx_hbm, out_hbm, idx_v, rows_a, rows_b, rows_c,
                sem_a, sem_b):
        wid = lax.axis_index("s") * _NC + lax.axis_index("c")
        base = wid * _BPW
        pltpu.sync_copy(idx_hbm.at[pl.ds(base, _BPW)], idx_v)
        bufs = (rows_a, rows_b)
        sems = (sem_a, sem_b)

        def _issue(j):
            return pltpu.async_copy(
                table_hbm.at[idx_v.at[pl.ds(j * _CHUNK, _CHUNK)]],
                bufs[j % 2], sems[j % 2])

        cps = {0: _issue(0)}
        for j in range(_NCH):
            if j + 1 < _NCH:
                cps[j + 1] = _issue(j + 1)
            cps[j].wait()
            buf = bufs[j % 2]

            def _compact(b, j=j, buf=buf):
                for c2 in range(BIT_WIDTH // 16):
                    rows_c[j * _CHUNK + b, pl.ds(c2 * 16, 16)] = (
                        buf[b, pl.ds(c2 * 16, 16)])

            plsc.parallel_loop(0, _CHUNK, unroll=8)(_compact)

        pltpu.sync_copy(rows_c, out_hbm.at[pl.ds(base, _BPW)])

    return _gather


def kernel(idx, register_keys, W1, b1, W2, b2, temperature, register_values):
    inv_temp = 1.0 / jnp.maximum(jnp.abs(temperature), 0.1)
    keys_scaled = register_keys * inv_temp
    w1p = jnp.zeros((8, KEY_DIM), jnp.float32).at[:5, :].set(W1)
    table = _build_table(keys_scaled, w1p, b1.reshape(1, KEY_DIM), W2,
                         b2.reshape(1, KEY_DIM), register_values)
    return _gather_kernel()(table, idx.astype(jnp.int32))


# speedup vs baseline: 1.6612x; 1.6612x over previous
import functools

import jax
import jax.numpy as jnp
from jax import lax
from jax.experimental import pallas as pl
from jax.experimental.pallas import tpu as pltpu
from jax.experimental.pallas import tpu_sc as plsc

N_REGS = 32
BIT_WIDTH = 64
KEY_DIM = 128
BATCH = 16384

_NC = 2
_NS = 16
_NW = _NC * _NS
_BPW = BATCH // _NW
_CHUNK = 128
_NCH = _BPW // _CHUNK


def _table_body(keys_ref, w1_ref, b1_ref, w2_ref, b2_ref, vals_ref, out_ref):
    r = lax.broadcasted_iota(jnp.int32, (N_REGS, 8), 0)
    c = lax.broadcasted_iota(jnp.int32, (N_REGS, 8), 1)
    bits = jnp.where(c < 5, (r >> c) & 1, 0).astype(jnp.float32)
    h = jnp.dot(bits, w1_ref[...], preferred_element_type=jnp.float32) + b1_ref[...]
    h = 0.5 * h * (1.0 + lax.erf(h * (2.0 ** -0.5)))
    q = jnp.dot(h, w2_ref[...], preferred_element_type=jnp.float32) + b2_ref[...]
    sim = lax.dot_general(q, keys_ref[...], (((1,), (1,)), ((), ())),
                          preferred_element_type=jnp.float32)
    m = jnp.max(sim, axis=1, keepdims=True)
    e = jnp.exp(sim - m)
    attn = e / jnp.sum(e, axis=1, keepdims=True)
    tab = jnp.dot(attn, vals_ref[...], preferred_element_type=jnp.float32)
    row = lax.broadcasted_iota(jnp.int32, (N_REGS, BIT_WIDTH), 0)
    tab = jnp.where(row == N_REGS - 1, 0.0, tab)
    out_ref[...] = jnp.concatenate(
        [tab, jnp.zeros((N_REGS, BIT_WIDTH), jnp.float32)], axis=1)


def _build_table(keys_scaled, w1p, b1, w2, b2, vals):
    return pl.pallas_call(
        _table_body,
        out_shape=jax.ShapeDtypeStruct((N_REGS, 2 * BIT_WIDTH), jnp.float32),
    )(keys_scaled, w1p, b1, w2, b2, vals)


@functools.cache
def _gather_kernel():
    mesh = plsc.VectorSubcoreMesh(core_axis_name="c", subcore_axis_name="s")

    @functools.partial(
        pl.kernel,
        mesh=mesh,
        out_type=jax.ShapeDtypeStruct((BATCH, BIT_WIDTH), jnp.float32),
        scratch_types=[
            pltpu.VMEM((_BPW,), jnp.int32),
            pltpu.VMEM((_CHUNK, 2 * BIT_WIDTH), jnp.float32),
            pltpu.VMEM((_CHUNK, 2 * BIT_WIDTH), jnp.float32),
            pltpu.VMEM((_BPW, BIT_WIDTH), jnp.float32),
            pltpu.VMEM_SHARED((N_REGS, 2 * BIT_WIDTH), jnp.float32),
            pltpu.SemaphoreType.DMA,
            pltpu.SemaphoreType.DMA,
        ],
    )
    def _gather(table_hbm, idx_hbm, out_hbm, idx_v, rows_a, rows_b, rows_c,
                table_sh, sem_a, sem_b):
        s = lax.axis_index("s")
        wid = s * _NC + lax.axis_index("c")
        base = wid * _BPW

        @pl.when(s == 0)
        def _():
            pltpu.sync_copy(table_hbm, table_sh)

        pltpu.sync_copy(idx_hbm.at[pl.ds(base, _BPW)], idx_v)
        plsc.subcore_barrier()
        bufs = (rows_a, rows_b)
        sems = (sem_a, sem_b)

        def _issue(j):
            return pltpu.async_copy(
                table_sh.at[idx_v.at[pl.ds(j * _CHUNK, _CHUNK)]],
                bufs[j % 2], sems[j % 2])

        cps = {0: _issue(0)}
        for j in range(_NCH):
            if j + 1 < _NCH:
                cps[j + 1] = _issue(j + 1)
            cps[j].wait()
            buf = bufs[j % 2]

            def _compact(b, j=j, buf=buf):
                for c2 in range(BIT_WIDTH // 16):
                    rows_c[j * _CHUNK + b, pl.ds(c2 * 16, 16)] = (
                        buf[b, pl.ds(c2 * 16, 16)])

            plsc.parallel_loop(0, _CHUNK, unroll=8)(_compact)

        pltpu.sync_copy(rows_c, out_hbm.at[pl.ds(base, _BPW)])

    return _gather


def kernel(idx, register_keys, W1, b1, W2, b2, temperature, register_values):
    inv_temp = 1.0 / jnp.maximum(jnp.abs(temperature), 0.1)
    keys_scaled = register_keys * inv_temp
    w1p = jnp.zeros((8, KEY_DIM), jnp.float32).at[:5, :].set(W1)
    table = _build_table(keys_scaled, w1p, b1.reshape(1, KEY_DIM), W2,
                         b2.reshape(1, KEY_DIM), register_values)
    return _gather_kernel()(table, idx.astype(jnp.int32))


# speedup vs baseline: 1.7001x; 1.0234x over previous
import functools

import jax
import jax.numpy as jnp
from jax import lax
from jax.experimental import pallas as pl
from jax.experimental.pallas import tpu as pltpu
from jax.experimental.pallas import tpu_sc as plsc

N_REGS = 32
BIT_WIDTH = 64
KEY_DIM = 128
BATCH = 16384

_NC = 2
_NS = 16
_NW = _NC * _NS
_BPW = BATCH // _NW
_CHUNK = 128
_NCH = _BPW // _CHUNK


def _table_body(temp_ref, keys_ref, w1_ref, b1_ref, w2_ref, b2_ref, vals_ref,
                out_ref):
    r = lax.broadcasted_iota(jnp.int32, (N_REGS, 8), 0)
    c = lax.broadcasted_iota(jnp.int32, (N_REGS, 8), 1)
    bits = jnp.where(c < 5, (r >> c) & 1, 0).astype(jnp.float32)
    w1 = jnp.where(lax.broadcasted_iota(jnp.int32, (8, KEY_DIM), 0) < 5,
                   w1_ref[...], 0.0)
    h = jnp.dot(bits, w1, preferred_element_type=jnp.float32) + b1_ref[...]
    h = 0.5 * h * (1.0 + lax.erf(h * (2.0 ** -0.5)))
    q = jnp.dot(h, w2_ref[...], preferred_element_type=jnp.float32) + b2_ref[...]
    sim = lax.dot_general(q, keys_ref[...], (((1,), (1,)), ((), ())),
                          preferred_element_type=jnp.float32)
    inv_temp = 1.0 / jnp.maximum(jnp.abs(temp_ref[0]), 0.1)
    sim = sim * inv_temp
    m = jnp.max(sim, axis=1, keepdims=True)
    e = jnp.exp(sim - m)
    attn = e / jnp.sum(e, axis=1, keepdims=True)
    tab = jnp.dot(attn, vals_ref[...], preferred_element_type=jnp.float32)
    row = lax.broadcasted_iota(jnp.int32, (N_REGS, BIT_WIDTH), 0)
    tab = jnp.where(row == N_REGS - 1, 0.0, tab)
    out_ref[...] = jnp.concatenate(
        [tab, jnp.zeros((N_REGS, BIT_WIDTH), jnp.float32)], axis=1)


def _build_table(temperature, keys, w1, b1, w2, b2, vals):
    return pl.pallas_call(
        _table_body,
        out_shape=jax.ShapeDtypeStruct((N_REGS, 2 * BIT_WIDTH), jnp.float32),
        in_specs=[pl.BlockSpec(memory_space=pltpu.SMEM)] +
                 [pl.BlockSpec(memory_space=pltpu.VMEM)] * 6,
    )(temperature, keys, w1, b1, w2, b2, vals)


@functools.cache
def _gather_kernel():
    mesh = plsc.VectorSubcoreMesh(core_axis_name="c", subcore_axis_name="s")

    @functools.partial(
        pl.kernel,
        mesh=mesh,
        out_type=jax.ShapeDtypeStruct((BATCH, BIT_WIDTH), jnp.float32),
        scratch_types=[
            pltpu.VMEM((_BPW,), jnp.int32),
            pltpu.VMEM((_CHUNK, 2 * BIT_WIDTH), jnp.float32),
            pltpu.VMEM((_CHUNK, 2 * BIT_WIDTH), jnp.float32),
            pltpu.VMEM((_BPW, BIT_WIDTH), jnp.float32),
            pltpu.VMEM_SHARED((N_REGS, 2 * BIT_WIDTH), jnp.float32),
            pltpu.SemaphoreType.DMA,
            pltpu.SemaphoreType.DMA,
        ],
    )
    def _gather(table_hbm, idx_hbm, out_hbm, idx_v, rows_a, rows_b, rows_c,
                table_sh, sem_a, sem_b):
        s = lax.axis_index("s")
        wid = s * _NC + lax.axis_index("c")
        base = wid * _BPW

        @pl.when(s == 0)
        def _():
            pltpu.sync_copy(table_hbm, table_sh)

        pltpu.sync_copy(idx_hbm.at[pl.ds(base, _BPW)], idx_v)
        plsc.subcore_barrier()
        bufs = (rows_a, rows_b)
        sems = (sem_a, sem_b)

        def _issue(j):
            return pltpu.async_copy(
                table_sh.at[idx_v.at[pl.ds(j * _CHUNK, _CHUNK)]],
                bufs[j % 2], sems[j % 2])

        cps = {0: _issue(0)}
        for j in range(_NCH):
            if j + 1 < _NCH:
                cps[j + 1] = _issue(j + 1)
            cps[j].wait()
            buf = bufs[j % 2]

            def _compact(b, j=j, buf=buf):
                for c2 in range(BIT_WIDTH // 16):
                    rows_c[j * _CHUNK + b, pl.ds(c2 * 16, 16)] = (
                        buf[b, pl.ds(c2 * 16, 16)])

            plsc.parallel_loop(0, _CHUNK, unroll=8)(_compact)

        pltpu.sync_copy(rows_c, out_hbm.at[pl.ds(base, _BPW)])

    return _gather


def kernel(idx, register_keys, W1, b1, W2, b2, temperature, register_values):
    w1p = jnp.concatenate([W1, W1[:3]], axis=0)
    table = _build_table(temperature.reshape(1), register_keys, w1p,
                         b1.reshape(1, KEY_DIM), W2, b2.reshape(1, KEY_DIM),
                         register_values)
    return _gather_kernel()(table, idx.astype(jnp.int32))
